# Initial kernel scaffold; baseline (speedup 1.0000x reference)
import jax
import jax.numpy as jnp
from jax.experimental import pallas as pl


def kernel(x, pos, edge_index, edge_attr, batch_indices, We1_0, be1_0, We2_0, be2_0, Wh1_0, bh1_0, Wh2_0, bh2_0, We1_1, be1_1, We2_1, be2_1, Wh1_1, bh1_1, Wh2_1, bh2_1, Wm1, bm1, Wm2, bm2, Wl, bl):
    raise NotImplementedError("write your pallas kernel here")


# speedup vs baseline: 1.6800x; 1.6800x over previous
import functools

import jax
import jax.numpy as jnp
from jax import lax
from jax.experimental import pallas as pl
from jax.experimental.pallas import tpu as pltpu
from jax.experimental.pallas import tpu_sc as plsc

N = 10000
NPAD = 10240
E = 320000
D = 128
ED = 16
NB = 16

NC, NS = 2, 16
NW = NC * NS
CHUNK = 128
EPAD = 327680
ROWS = EPAD // CHUNK
RPW = ROWS // NW
RPC = ROWS // NC
NPT = NPAD // NS

_sc_mesh = plsc.VectorSubcoreMesh(core_axis_name="c", subcore_axis_name="s")


def _gather_body(with_d2, *refs):
    if with_d2:
        (a_hbm, b_hbm, px_hbm, py_hbm, pz_hbm, src_hbm, dst_hbm,
         g_hbm, d2_hbm, sidx, didx, bufa, bufb, posx, posy, posz, d2b) = refs
    else:
        (a_hbm, b_hbm, src_hbm, dst_hbm, g_hbm, sidx, didx, bufa, bufb) = refs

    c = lax.axis_index("c")
    s = lax.axis_index("s")
    w = s * NC + c
    row0 = w * RPW
    pltpu.sync_copy(src_hbm.at[pl.ds(row0, RPW)], sidx)
    pltpu.sync_copy(dst_hbm.at[pl.ds(row0, RPW)], didx)
    if with_d2:
        pltpu.sync_copy(px_hbm, posx)
        pltpu.sync_copy(py_hbm, posy)
        pltpu.sync_copy(pz_hbm, posz)

    def chunk(j, carry):
        pltpu.sync_copy(a_hbm.at[sidx.at[j]], bufa)
        pltpu.sync_copy(b_hbm.at[didx.at[j]], bufb)

        def addrow(r, carry2):
            for v in range(8):
                sl = pl.ds(v * 16, 16)
                bufa[r, sl] = bufa[r, sl] + bufb[r, sl]
            return carry2

        lax.fori_loop(0, CHUNK, addrow, 0)
        pltpu.sync_copy(bufa, g_hbm.at[pl.ds((row0 + j) * CHUNK, CHUNK)])

        if with_d2:
            for v in range(8):
                sl = pl.ds(v * 16, 16)
                sv = sidx[j, sl]
                dv = didx[j, sl]
                si = [lax.shift_right_logical(sv, 7), jnp.bitwise_and(sv, 127)]
                di = [lax.shift_right_logical(dv, 7), jnp.bitwise_and(dv, 127)]
                dx = plsc.load_gather(posx, si) - plsc.load_gather(posx, di)
                dy = plsc.load_gather(posy, si) - plsc.load_gather(posy, di)
                dz = plsc.load_gather(posz, si) - plsc.load_gather(posz, di)
                d2b[sl] = dx * dx + dy * dy + dz * dz
            pltpu.sync_copy(d2b, d2_hbm.at[pl.ds((row0 + j) * CHUNK, CHUNK)])
        return carry

    lax.fori_loop(0, RPW, chunk, 0)


def _make_gather(with_d2):
    outs = [jax.ShapeDtypeStruct((EPAD, D), jnp.float32)]
    scratch = [
        pltpu.VMEM((RPW, CHUNK), jnp.int32),
        pltpu.VMEM((RPW, CHUNK), jnp.int32),
        pltpu.VMEM((CHUNK, D), jnp.float32),
        pltpu.VMEM((CHUNK, D), jnp.float32),
    ]
    if with_d2:
        outs.append(jax.ShapeDtypeStruct((EPAD,), jnp.float32))
        scratch += [
            pltpu.VMEM((NPAD // CHUNK, CHUNK), jnp.float32),
            pltpu.VMEM((NPAD // CHUNK, CHUNK), jnp.float32),
            pltpu.VMEM((NPAD // CHUNK, CHUNK), jnp.float32),
            pltpu.VMEM((CHUNK,), jnp.float32),
        ]
    return pl.kernel(
        functools.partial(_gather_body, with_d2),
        out_type=tuple(outs),
        mesh=_sc_mesh,
        scratch_types=scratch,
        compiler_params=pltpu.CompilerParams(needs_layout_passes=False),
    )


_gather_d2 = _make_gather(True)
_gather = _make_gather(False)


def _scatter_body(m_hbm, dsts_hbm, out_hbm, didx, mbuf, agg):
    c = lax.axis_index("c")
    s = lax.axis_index("s")

    def zrow(r, carry):
        for v in range(8):
            mbuf[r, pl.ds(v * 16, 16)] = jnp.zeros((16,), jnp.float32)
        return carry

    lax.fori_loop(0, CHUNK, zrow, 0)
    for t in range(NPT // CHUNK):
        pltpu.sync_copy(mbuf, agg.at[pl.ds(s * NPT + t * CHUNK, CHUNK)])
    plsc.subcore_barrier()

    row0 = c * RPC + s * RPW
    pltpu.sync_copy(dsts_hbm.at[pl.ds(row0, RPW)], didx)

    def chunk(j, carry):
        pltpu.sync_copy(m_hbm.at[pl.ds((row0 + j) * CHUNK, CHUNK)], mbuf)
        pltpu.sync_copy(mbuf, agg.at[didx.at[j]], add=True)
        return carry

    lax.fori_loop(0, RPW, chunk, 0)
    plsc.subcore_barrier()
    pltpu.sync_copy(agg.at[pl.ds(s * NPT, NPT)],
                    out_hbm.at[c, pl.ds(s * NPT, NPT)])


_scatter = pl.kernel(
    _scatter_body,
    out_type=jax.ShapeDtypeStruct((NC, NPAD, D), jnp.float32),
    mesh=_sc_mesh,
    scratch_types=[
        pltpu.VMEM((RPW, CHUNK), jnp.int32),
        pltpu.VMEM((CHUNK, D), jnp.float32),
        pltpu.VMEM_SHARED((NPAD, D), jnp.float32),
    ],
    compiler_params=pltpu.CompilerParams(needs_layout_passes=False),
)


def _silu(x):
    return x * jax.nn.sigmoid(x)


def _mm2_body(h_ref, wa_ref, wb_ref, a_ref, b_ref):
    h = h_ref[...]
    a_ref[...] = jnp.dot(h, wa_ref[...], preferred_element_type=jnp.float32)
    b_ref[...] = jnp.dot(h, wb_ref[...], preferred_element_type=jnp.float32)


def _mm2(h, wa, wb):
    bn = 512
    grid = NPAD // bn
    return pl.pallas_call(
        _mm2_body,
        grid=(grid,),
        in_specs=[
            pl.BlockSpec((bn, D), lambda i: (i, 0)),
            pl.BlockSpec((D, D), lambda i: (0, 0)),
            pl.BlockSpec((D, D), lambda i: (0, 0)),
        ],
        out_specs=[
            pl.BlockSpec((bn, D), lambda i: (i, 0)),
            pl.BlockSpec((bn, D), lambda i: (i, 0)),
        ],
        out_shape=[
            jax.ShapeDtypeStruct((NPAD, D), jnp.float32),
            jax.ShapeDtypeStruct((NPAD, D), jnp.float32),
        ],
    )(h, wa, wb)


def _msg_body(g_ref, ee_ref, wd_ref, we2_ref, be2_ref, out_ref):
    z = g_ref[...] + jnp.dot(ee_ref[...], wd_ref[...],
                             preferred_element_type=jnp.float32)
    m = _silu(z)
    y = jnp.dot(m, we2_ref[...], preferred_element_type=jnp.float32) + be2_ref[...]
    out_ref[...] = _silu(y)


def _msg(g, ee, wd, we2, be2):
    be = 1024
    grid = EPAD // be
    ec = ee.shape[1]
    return pl.pallas_call(
        _msg_body,
        grid=(grid,),
        in_specs=[
            pl.BlockSpec((be, D), lambda i: (i, 0)),
            pl.BlockSpec((be, ec), lambda i: (i, 0)),
            pl.BlockSpec((ec, D), lambda i: (0, 0)),
            pl.BlockSpec((D, D), lambda i: (0, 0)),
            pl.BlockSpec((1, D), lambda i: (0, 0)),
        ],
        out_specs=pl.BlockSpec((be, D), lambda i: (i, 0)),
        out_shape=jax.ShapeDtypeStruct((EPAD, D), jnp.float32),
    )(g, ee, wd, we2, be2)


def _upd_body(next_ab, *refs):
    if next_ab:
        (h_ref, p0_ref, p1_ref, w1h_ref, w1a_ref, b1_ref, w2_ref, b2_ref,
         wa_ref, wb_ref, hn_ref, a_ref, b_ref) = refs
    else:
        (h_ref, p0_ref, p1_ref, w1h_ref, w1a_ref, b1_ref, w2_ref, b2_ref,
         hn_ref) = refs
    h = h_ref[...]
    agg = p0_ref[...] + p1_ref[...]
    t = (jnp.dot(h, w1h_ref[...], preferred_element_type=jnp.float32)
         + jnp.dot(agg, w1a_ref[...], preferred_element_type=jnp.float32)
         + b1_ref[...])
    t = _silu(t)
    hn = jnp.dot(t, w2_ref[...], preferred_element_type=jnp.float32) + b2_ref[...]
    hn_ref[...] = hn
    if next_ab:
        a_ref[...] = jnp.dot(hn, wa_ref[...], preferred_element_type=jnp.float32)
        b_ref[...] = jnp.dot(hn, wb_ref[...], preferred_element_type=jnp.float32)


def _update(h, p0, p1, w1h, w1a, b1, w2, b2, wa=None, wb=None):
    bn = 512
    grid = NPAD // bn
    next_ab = wa is not None
    blk = pl.BlockSpec((bn, D), lambda i: (i, 0))
    wblk = pl.BlockSpec((D, D), lambda i: (0, 0))
    bblk = pl.BlockSpec((1, D), lambda i: (0, 0))
    in_specs = [blk, blk, blk, wblk, wblk, bblk, wblk, bblk]
    args = [h, p0, p1, w1h, w1a, b1, w2, b2]
    out_specs = [blk]
    out_shape = [jax.ShapeDtypeStruct((NPAD, D), jnp.float32)]
    if next_ab:
        in_specs += [wblk, wblk]
        args += [wa, wb]
        out_specs += [blk, blk]
        out_shape += [jax.ShapeDtypeStruct((NPAD, D), jnp.float32),
                      jax.ShapeDtypeStruct((NPAD, D), jnp.float32)]
    return pl.pallas_call(
        functools.partial(_upd_body, next_ab),
        grid=(grid,),
        in_specs=in_specs,
        out_specs=out_specs,
        out_shape=out_shape,
    )(*args)


def _pool_body(h_ref, bi_ref, wm1_ref, bm1_ref, wm2_ref, bm2_ref, wl_ref,
               bl_ref, out_ref, sum_ref, cnt_ref):
    i = pl.program_id(0)

    @pl.when(i == 0)
    def _():
        sum_ref[...] = jnp.zeros_like(sum_ref)
        cnt_ref[...] = jnp.zeros_like(cnt_ref)

    bi = bi_ref[...]
    oh = (bi == lax.broadcasted_iota(jnp.int32, bi.shape, 1)).astype(jnp.float32)
    dn = (((0,), (0,)), ((), ()))
    sum_ref[...] += lax.dot_general(oh, h_ref[...], dn,
                                    preferred_element_type=jnp.float32)
    cnt_ref[...] += lax.dot_general(oh, jnp.ones_like(h_ref[...]), dn,
                                    preferred_element_type=jnp.float32)

    @pl.when(i == pl.num_programs(0) - 1)
    def _():
        pooled = sum_ref[...] / jnp.maximum(cnt_ref[...], 1.0)
        r = jnp.maximum(
            jnp.dot(pooled, wm1_ref[...], preferred_element_type=jnp.float32)
            + bm1_ref[...], 0.0)
        r = jnp.maximum(
            jnp.dot(r, wm2_ref[...], preferred_element_type=jnp.float32)
            + bm2_ref[...], 0.0)
        out_ref[...] = (jnp.dot(r, wl_ref[...], preferred_element_type=jnp.float32)
                        + bl_ref[...])


def _pool_head(h, bi16, wm1, bm1, wm2, bm2, wlp, blp):
    bn = 1024
    grid = NPAD // bn
    wblk = pl.BlockSpec((D, D), lambda i: (0, 0))
    bblk = pl.BlockSpec((1, D), lambda i: (0, 0))
    return pl.pallas_call(
        _pool_body,
        grid=(grid,),
        in_specs=[
            pl.BlockSpec((bn, D), lambda i: (i, 0)),
            pl.BlockSpec((bn, NB), lambda i: (i, 0)),
            wblk, bblk, wblk, bblk, wblk, bblk,
        ],
        out_specs=pl.BlockSpec((NB, D), lambda i: (0, 0)),
        out_shape=jax.ShapeDtypeStruct((NB, D), jnp.float32),
        scratch_shapes=[
            pltpu.VMEM((NB, D), jnp.float32),
            pltpu.VMEM((NB, D), jnp.float32),
        ],
    )(h, bi16, wm1, bm1, wm2, bm2, wlp, blp)


def kernel(x, pos, edge_index, edge_attr, batch_indices,
           We1_0, be1_0, We2_0, be2_0, Wh1_0, bh1_0, Wh2_0, bh2_0,
           We1_1, be1_1, We2_1, be2_1, Wh1_1, bh1_1, Wh2_1, bh2_1,
           Wm1, bm1, Wm2, bm2, Wl, bl):
    f32 = jnp.float32
    xp = jnp.pad(x, ((0, NPAD - N), (0, 0)))
    px = jnp.pad(pos[:, 0], (0, NPAD - N)).reshape(NPAD // CHUNK, CHUNK)
    py = jnp.pad(pos[:, 1], (0, NPAD - N)).reshape(NPAD // CHUNK, CHUNK)
    pz = jnp.pad(pos[:, 2], (0, NPAD - N)).reshape(NPAD // CHUNK, CHUNK)
    src = jnp.pad(edge_index[0], (0, EPAD - E)).reshape(ROWS, CHUNK)
    dst = jnp.pad(edge_index[1], (0, EPAD - E)).reshape(ROWS, CHUNK)
    dsts = jnp.pad(edge_index[1], (0, EPAD - E),
                   constant_values=N).reshape(ROWS, CHUNK)
    eap = jnp.pad(edge_attr, ((0, EPAD - E), (0, 0)))
    bi16 = jnp.broadcast_to(
        jnp.pad(batch_indices, (0, NPAD - N), constant_values=NB)[:, None],
        (NPAD, NB))

    wd0 = jnp.concatenate([We1_0[2 * D + 1:], We1_0[2 * D:2 * D + 1],
                           be1_0[None]], axis=0)
    wd1 = jnp.concatenate([We1_1[2 * D + 1:], We1_1[2 * D:2 * D + 1],
                           be1_1[None]], axis=0)
    wlp = jnp.pad(Wl, ((0, 0), (0, D - 1)))
    blp = jnp.pad(bl, (0, D - 1))[None]

    a0, b0 = _mm2(xp, We1_0[:D], We1_0[D:2 * D])
    g0, d2 = _gather_d2(a0, b0, px, py, pz, src, dst)
    ee = jnp.concatenate([eap, d2[:, None], jnp.ones((EPAD, 1), f32)], axis=1)
    m0 = _msg(g0, ee, wd0, We2_0, be2_0[None])
    p = _scatter(m0, dsts)
    h1, a1, b1 = _update(xp, p[0], p[1], Wh1_0[:D], Wh1_0[D:], bh1_0[None],
                         Wh2_0, bh2_0[None], We1_1[:D], We1_1[D:2 * D])
    g1, = _gather(a1, b1, src, dst)
    m1 = _msg(g1, ee, wd1, We2_1, be2_1[None])
    p = _scatter(m1, dsts)
    h2 = _update(h1, p[0], p[1], Wh1_1[:D], Wh1_1[D:], bh1_1[None],
                 Wh2_1, bh2_1[None])[0]
    out = _pool_head(h2, bi16, Wm1, bm1[None], Wm2, bm2[None], wlp, blp)
    return out[:, :1]


# speedup vs baseline: 2.6630x; 1.5851x over previous
import functools

import jax
import jax.numpy as jnp
from jax import lax
from jax.experimental import pallas as pl
from jax.experimental.pallas import tpu as pltpu
from jax.experimental.pallas import tpu_sc as plsc

N = 10000
NPAD = 10240
E = 320000
D = 128
ED = 16
NB = 16

NC, NS = 2, 16
NW = NC * NS
CHUNK = 128
EPAD = 327680
ROWS = EPAD // CHUNK
RPW = ROWS // NW
RPC = ROWS // NC
NPT = NPAD // NS

_sc_mesh = plsc.VectorSubcoreMesh(core_axis_name="c", subcore_axis_name="s")


def _gather_body(with_d2, *refs):
    if with_d2:
        (a_hbm, b_hbm, px_hbm, py_hbm, pz_hbm, src_hbm, dst_hbm,
         g_hbm, d2_hbm, sidx, didx, ba0, ba1, bb0, bb1,
         posx, posy, posz, d2b, sa0, sa1, sb0, sb1, sw0, sw1) = refs
    else:
        (a_hbm, b_hbm, src_hbm, dst_hbm, g_hbm, sidx, didx,
         ba0, ba1, bb0, bb1, sa0, sa1, sb0, sb1, sw0, sw1) = refs
    bufa = (ba0, ba1)
    bufb = (bb0, bb1)
    sa = (sa0, sa1)
    sb = (sb0, sb1)
    sw = (sw0, sw1)

    c = lax.axis_index("c")
    s = lax.axis_index("s")
    w = s * NC + c
    row0 = w * RPW
    pltpu.sync_copy(src_hbm.at[pl.ds(row0, RPW)], sidx)
    pltpu.sync_copy(dst_hbm.at[pl.ds(row0, RPW)], didx)
    if with_d2:
        pltpu.sync_copy(px_hbm, posx)
        pltpu.sync_copy(py_hbm, posy)
        pltpu.sync_copy(pz_hbm, posz)

    def start(j, t):
        pltpu.async_copy(a_hbm.at[sidx.at[j]], bufa[t], sa[t])
        pltpu.async_copy(b_hbm.at[didx.at[j]], bufb[t], sb[t])

    def wait_gather(t):
        pltpu.make_async_copy(a_hbm.at[sidx.at[0]], bufa[t], sa[t]).wait()
        pltpu.make_async_copy(b_hbm.at[didx.at[0]], bufb[t], sb[t]).wait()

    def wait_write(t):
        pltpu.make_async_copy(
            bufa[t], g_hbm.at[pl.ds(row0 * CHUNK, CHUNK)], sw[t]).wait()

    start(0, 0)

    def body2(i, carry):
        for t in (0, 1):
            j = i * 2 + t
            nt = 1 - t

            @pl.when(jnp.logical_and(j >= 1, j + 1 < RPW))
            def _():
                wait_write(nt)

            @pl.when(j + 1 < RPW)
            def _():
                start(j + 1, nt)

            if with_d2:
                for v in range(8):
                    sl = pl.ds(v * 16, 16)
                    sv = sidx[j, sl]
                    dv = didx[j, sl]
                    si = [lax.shift_right_logical(sv, 7),
                          jnp.bitwise_and(sv, 127)]
                    di = [lax.shift_right_logical(dv, 7),
                          jnp.bitwise_and(dv, 127)]
                    dx = plsc.load_gather(posx, si) - plsc.load_gather(posx, di)
                    dy = plsc.load_gather(posy, si) - plsc.load_gather(posy, di)
                    dz = plsc.load_gather(posz, si) - plsc.load_gather(posz, di)
                    d2b[sl] = dx * dx + dy * dy + dz * dz
                pltpu.sync_copy(d2b,
                                d2_hbm.at[pl.ds((row0 + j) * CHUNK, CHUNK)])

            wait_gather(t)

            def addrow(r, carry2):
                for v in range(8):
                    sl = pl.ds(v * 16, 16)
                    bufa[t][r, sl] = bufa[t][r, sl] + bufb[t][r, sl]
                return carry2

            lax.fori_loop(0, CHUNK, addrow, 0)
            pltpu.async_copy(bufa[t],
                             g_hbm.at[pl.ds((row0 + j) * CHUNK, CHUNK)], sw[t])
        return carry

    lax.fori_loop(0, RPW // 2, body2, 0)
    wait_write(0)
    wait_write(1)


def _make_gather(with_d2):
    outs = [jax.ShapeDtypeStruct((EPAD, D), jnp.float32)]
    scratch = [
        pltpu.VMEM((RPW, CHUNK), jnp.int32),
        pltpu.VMEM((RPW, CHUNK), jnp.int32),
        pltpu.VMEM((CHUNK, D), jnp.float32),
        pltpu.VMEM((CHUNK, D), jnp.float32),
        pltpu.VMEM((CHUNK, D), jnp.float32),
        pltpu.VMEM((CHUNK, D), jnp.float32),
    ]
    if with_d2:
        outs.append(jax.ShapeDtypeStruct((EPAD,), jnp.float32))
        scratch += [
            pltpu.VMEM((NPAD // CHUNK, CHUNK), jnp.float32),
            pltpu.VMEM((NPAD // CHUNK, CHUNK), jnp.float32),
            pltpu.VMEM((NPAD // CHUNK, CHUNK), jnp.float32),
            pltpu.VMEM((CHUNK,), jnp.float32),
        ]
    scratch += [pltpu.SemaphoreType.DMA] * 6
    return pl.kernel(
        functools.partial(_gather_body, with_d2),
        out_type=tuple(outs),
        mesh=_sc_mesh,
        scratch_types=scratch,
        compiler_params=pltpu.CompilerParams(needs_layout_passes=False),
    )


_gather_d2 = _make_gather(True)
_gather = _make_gather(False)


def _scatter_body(m_hbm, dsts_hbm, out_hbm, didx, mb0, mb1, agg, sr0, sr1):
    c = lax.axis_index("c")
    s = lax.axis_index("s")
    mbuf = (mb0, mb1)
    sr = (sr0, sr1)

    def zrow(r, carry):
        for v in range(8):
            mb0[r, pl.ds(v * 16, 16)] = jnp.zeros((16,), jnp.float32)
        return carry

    lax.fori_loop(0, CHUNK, zrow, 0)
    for t in range(NPT // CHUNK):
        pltpu.sync_copy(mb0, agg.at[pl.ds(s * NPT + t * CHUNK, CHUNK)])
    plsc.subcore_barrier()

    row0 = c * RPC + s * RPW
    pltpu.sync_copy(dsts_hbm.at[pl.ds(row0, RPW)], didx)

    def start(j, t):
        pltpu.async_copy(m_hbm.at[pl.ds((row0 + j) * CHUNK, CHUNK)],
                         mbuf[t], sr[t])

    def wait_read(t):
        pltpu.make_async_copy(m_hbm.at[pl.ds(row0 * CHUNK, CHUNK)],
                              mbuf[t], sr[t]).wait()

    start(0, 0)

    def body2(i, carry):
        for t in (0, 1):
            j = i * 2 + t

            @pl.when(j + 1 < RPW)
            def _():
                start(j + 1, 1 - t)

            wait_read(t)
            pltpu.sync_copy(mbuf[t], agg.at[didx.at[j]], add=True)
        return carry

    lax.fori_loop(0, RPW // 2, body2, 0)
    plsc.subcore_barrier()
    pltpu.sync_copy(agg.at[pl.ds(s * NPT, NPT)],
                    out_hbm.at[c, pl.ds(s * NPT, NPT)])


_scatter = pl.kernel(
    _scatter_body,
    out_type=jax.ShapeDtypeStruct((NC, NPAD, D), jnp.float32),
    mesh=_sc_mesh,
    scratch_types=[
        pltpu.VMEM((RPW, CHUNK), jnp.int32),
        pltpu.VMEM((CHUNK, D), jnp.float32),
        pltpu.VMEM((CHUNK, D), jnp.float32),
        pltpu.VMEM_SHARED((NPAD, D), jnp.float32),
        pltpu.SemaphoreType.DMA,
        pltpu.SemaphoreType.DMA,
    ],
    compiler_params=pltpu.CompilerParams(needs_layout_passes=False),
)


def _silu(x):
    return x * jax.nn.sigmoid(x)


def _mm2_body(h_ref, wa_ref, wb_ref, a_ref, b_ref):
    h = h_ref[...]
    a_ref[...] = jnp.dot(h, wa_ref[...], preferred_element_type=jnp.float32)
    b_ref[...] = jnp.dot(h, wb_ref[...], preferred_element_type=jnp.float32)


def _mm2(h, wa, wb):
    bn = 512
    grid = NPAD // bn
    return pl.pallas_call(
        _mm2_body,
        grid=(grid,),
        in_specs=[
            pl.BlockSpec((bn, D), lambda i: (i, 0)),
            pl.BlockSpec((D, D), lambda i: (0, 0)),
            pl.BlockSpec((D, D), lambda i: (0, 0)),
        ],
        out_specs=[
            pl.BlockSpec((bn, D), lambda i: (i, 0)),
            pl.BlockSpec((bn, D), lambda i: (i, 0)),
        ],
        out_shape=[
            jax.ShapeDtypeStruct((NPAD, D), jnp.float32),
            jax.ShapeDtypeStruct((NPAD, D), jnp.float32),
        ],
    )(h, wa, wb)


def _msg_body(g_ref, ee_ref, wd_ref, we2_ref, be2_ref, out_ref):
    z = g_ref[...] + jnp.dot(ee_ref[...], wd_ref[...],
                             preferred_element_type=jnp.float32)
    m = _silu(z)
    y = jnp.dot(m, we2_ref[...], preferred_element_type=jnp.float32) + be2_ref[...]
    out_ref[...] = _silu(y)


def _msg(g, ee, wd, we2, be2):
    be = 1024
    grid = EPAD // be
    ec = ee.shape[1]
    return pl.pallas_call(
        _msg_body,
        grid=(grid,),
        in_specs=[
            pl.BlockSpec((be, D), lambda i: (i, 0)),
            pl.BlockSpec((be, ec), lambda i: (i, 0)),
            pl.BlockSpec((ec, D), lambda i: (0, 0)),
            pl.BlockSpec((D, D), lambda i: (0, 0)),
            pl.BlockSpec((1, D), lambda i: (0, 0)),
        ],
        out_specs=pl.BlockSpec((be, D), lambda i: (i, 0)),
        out_shape=jax.ShapeDtypeStruct((EPAD, D), jnp.float32),
    )(g, ee, wd, we2, be2)


def _upd_body(next_ab, *refs):
    if next_ab:
        (h_ref, p0_ref, p1_ref, w1h_ref, w1a_ref, b1_ref, w2_ref, b2_ref,
         wa_ref, wb_ref, hn_ref, a_ref, b_ref) = refs
    else:
        (h_ref, p0_ref, p1_ref, w1h_ref, w1a_ref, b1_ref, w2_ref, b2_ref,
         hn_ref) = refs
    h = h_ref[...]
    agg = p0_ref[...] + p1_ref[...]
    t = (jnp.dot(h, w1h_ref[...], preferred_element_type=jnp.float32)
         + jnp.dot(agg, w1a_ref[...], preferred_element_type=jnp.float32)
         + b1_ref[...])
    t = _silu(t)
    hn = jnp.dot(t, w2_ref[...], preferred_element_type=jnp.float32) + b2_ref[...]
    hn_ref[...] = hn
    if next_ab:
        a_ref[...] = jnp.dot(hn, wa_ref[...], preferred_element_type=jnp.float32)
        b_ref[...] = jnp.dot(hn, wb_ref[...], preferred_element_type=jnp.float32)


def _update(h, p0, p1, w1h, w1a, b1, w2, b2, wa=None, wb=None):
    bn = 512
    grid = NPAD // bn
    next_ab = wa is not None
    blk = pl.BlockSpec((bn, D), lambda i: (i, 0))
    wblk = pl.BlockSpec((D, D), lambda i: (0, 0))
    bblk = pl.BlockSpec((1, D), lambda i: (0, 0))
    in_specs = [blk, blk, blk, wblk, wblk, bblk, wblk, bblk]
    args = [h, p0, p1, w1h, w1a, b1, w2, b2]
    out_specs = [blk]
    out_shape = [jax.ShapeDtypeStruct((NPAD, D), jnp.float32)]
    if next_ab:
        in_specs += [wblk, wblk]
        args += [wa, wb]
        out_specs += [blk, blk]
        out_shape += [jax.ShapeDtypeStruct((NPAD, D), jnp.float32),
                      jax.ShapeDtypeStruct((NPAD, D), jnp.float32)]
    return pl.pallas_call(
        functools.partial(_upd_body, next_ab),
        grid=(grid,),
        in_specs=in_specs,
        out_specs=out_specs,
        out_shape=out_shape,
    )(*args)


def _pool_body(h_ref, bi_ref, wm1_ref, bm1_ref, wm2_ref, bm2_ref, wl_ref,
               bl_ref, out_ref, sum_ref, cnt_ref):
    i = pl.program_id(0)

    @pl.when(i == 0)
    def _():
        sum_ref[...] = jnp.zeros_like(sum_ref)
        cnt_ref[...] = jnp.zeros_like(cnt_ref)

    bi = bi_ref[...]
    oh = (bi == lax.broadcasted_iota(jnp.int32, bi.shape, 1)).astype(jnp.float32)
    dn = (((0,), (0,)), ((), ()))
    sum_ref[...] += lax.dot_general(oh, h_ref[...], dn,
                                    preferred_element_type=jnp.float32)
    cnt_ref[...] += lax.dot_general(oh, jnp.ones_like(h_ref[...]), dn,
                                    preferred_element_type=jnp.float32)

    @pl.when(i == pl.num_programs(0) - 1)
    def _():
        pooled = sum_ref[...] / jnp.maximum(cnt_ref[...], 1.0)
        r = jnp.maximum(
            jnp.dot(pooled, wm1_ref[...], preferred_element_type=jnp.float32)
            + bm1_ref[...], 0.0)
        r = jnp.maximum(
            jnp.dot(r, wm2_ref[...], preferred_element_type=jnp.float32)
            + bm2_ref[...], 0.0)
        out_ref[...] = (jnp.dot(r, wl_ref[...], preferred_element_type=jnp.float32)
                        + bl_ref[...])


def _pool_head(h, bi16, wm1, bm1, wm2, bm2, wlp, blp):
    bn = 1024
    grid = NPAD // bn
    wblk = pl.BlockSpec((D, D), lambda i: (0, 0))
    bblk = pl.BlockSpec((1, D), lambda i: (0, 0))
    return pl.pallas_call(
        _pool_body,
        grid=(grid,),
        in_specs=[
            pl.BlockSpec((bn, D), lambda i: (i, 0)),
            pl.BlockSpec((bn, NB), lambda i: (i, 0)),
            wblk, bblk, wblk, bblk, wblk, bblk,
        ],
        out_specs=pl.BlockSpec((NB, D), lambda i: (0, 0)),
        out_shape=jax.ShapeDtypeStruct((NB, D), jnp.float32),
        scratch_shapes=[
            pltpu.VMEM((NB, D), jnp.float32),
            pltpu.VMEM((NB, D), jnp.float32),
        ],
    )(h, bi16, wm1, bm1, wm2, bm2, wlp, blp)


def kernel(x, pos, edge_index, edge_attr, batch_indices,
           We1_0, be1_0, We2_0, be2_0, Wh1_0, bh1_0, Wh2_0, bh2_0,
           We1_1, be1_1, We2_1, be2_1, Wh1_1, bh1_1, Wh2_1, bh2_1,
           Wm1, bm1, Wm2, bm2, Wl, bl):
    f32 = jnp.float32
    xp = jnp.pad(x, ((0, NPAD - N), (0, 0)))
    px = jnp.pad(pos[:, 0], (0, NPAD - N)).reshape(NPAD // CHUNK, CHUNK)
    py = jnp.pad(pos[:, 1], (0, NPAD - N)).reshape(NPAD // CHUNK, CHUNK)
    pz = jnp.pad(pos[:, 2], (0, NPAD - N)).reshape(NPAD // CHUNK, CHUNK)
    src = jnp.pad(edge_index[0], (0, EPAD - E)).reshape(ROWS, CHUNK)
    dst = jnp.pad(edge_index[1], (0, EPAD - E)).reshape(ROWS, CHUNK)
    dsts = jnp.pad(edge_index[1], (0, EPAD - E),
                   constant_values=N).reshape(ROWS, CHUNK)
    eap = jnp.pad(edge_attr, ((0, EPAD - E), (0, 0)))
    bi16 = jnp.broadcast_to(
        jnp.pad(batch_indices, (0, NPAD - N), constant_values=NB)[:, None],
        (NPAD, NB))

    wd0 = jnp.concatenate([We1_0[2 * D + 1:], We1_0[2 * D:2 * D + 1],
                           be1_0[None]], axis=0)
    wd1 = jnp.concatenate([We1_1[2 * D + 1:], We1_1[2 * D:2 * D + 1],
                           be1_1[None]], axis=0)
    wlp = jnp.pad(Wl, ((0, 0), (0, D - 1)))
    blp = jnp.pad(bl, (0, D - 1))[None]

    a0, b0 = _mm2(xp, We1_0[:D], We1_0[D:2 * D])
    g0, d2 = _gather_d2(a0, b0, px, py, pz, src, dst)
    ee = jnp.concatenate([eap, d2[:, None], jnp.ones((EPAD, 1), f32)], axis=1)
    m0 = _msg(g0, ee, wd0, We2_0, be2_0[None])
    p = _scatter(m0, dsts)
    h1, a1, b1 = _update(xp, p[0], p[1], Wh1_0[:D], Wh1_0[D:], bh1_0[None],
                         Wh2_0, bh2_0[None], We1_1[:D], We1_1[D:2 * D])
    g1, = _gather(a1, b1, src, dst)
    m1 = _msg(g1, ee, wd1, We2_1, be2_1[None])
    p = _scatter(m1, dsts)
    h2 = _update(h1, p[0], p[1], Wh1_1[:D], Wh1_1[D:], bh1_1[None],
                 Wh2_1, bh2_1[None])[0]
    out = _pool_head(h2, bi16, Wm1, bm1[None], Wm2, bm2[None], wlp, blp)
    return out[:, :1]


# speedup vs baseline: 2.7890x; 1.0473x over previous
import functools

import jax
import jax.numpy as jnp
from jax import lax
from jax.experimental import pallas as pl
from jax.experimental.pallas import tpu as pltpu
from jax.experimental.pallas import tpu_sc as plsc

N = 10000
NPAD = 10240
E = 320000
D = 128
ED = 16
NB = 16

NC, NS = 2, 16
NW = NC * NS
CHUNK = 128
EPAD = 327680
ROWS = EPAD // CHUNK
RPW = ROWS // NW
RPC = ROWS // NC
NPT = NPAD // NS

_sc_mesh = plsc.VectorSubcoreMesh(core_axis_name="c", subcore_axis_name="s")


GC = 64
GPW = EPAD // NW // GC
GROWS = EPAD // GC


def _gather_body(with_d2, *refs):
    if with_d2:
        (a_hbm, b_hbm, px_hbm, py_hbm, pz_hbm, src_hbm, dst_hbm,
         g_hbm, d2_hbm, sidx, didx, ba0, ba1, ba2, bb0, bb1, bb2,
         posx, posy, posz, d2buf,
         sa0, sa1, sa2, sb0, sb1, sb2, sw0, sw1, sw2) = refs
    else:
        (a_hbm, b_hbm, src_hbm, dst_hbm, g_hbm, sidx, didx,
         ba0, ba1, ba2, bb0, bb1, bb2,
         sa0, sa1, sa2, sb0, sb1, sb2, sw0, sw1, sw2) = refs
    bufa = (ba0, ba1, ba2)
    bufb = (bb0, bb1, bb2)
    sa = (sa0, sa1, sa2)
    sb = (sb0, sb1, sb2)
    sw = (sw0, sw1, sw2)

    c = lax.axis_index("c")
    s = lax.axis_index("s")
    w = s * NC + c
    row0 = w * GPW
    pltpu.sync_copy(src_hbm.at[pl.ds(row0, GPW)], sidx)
    pltpu.sync_copy(dst_hbm.at[pl.ds(row0, GPW)], didx)
    if with_d2:
        pltpu.sync_copy(px_hbm, posx)
        pltpu.sync_copy(py_hbm, posy)
        pltpu.sync_copy(pz_hbm, posz)

    def start(j, t):
        pltpu.async_copy(a_hbm.at[sidx.at[j]], bufa[t], sa[t])
        pltpu.async_copy(b_hbm.at[didx.at[j]], bufb[t], sb[t])

    def wait_gather(t):
        pltpu.make_async_copy(a_hbm.at[sidx.at[0]], bufa[t], sa[t]).wait()
        pltpu.make_async_copy(b_hbm.at[didx.at[0]], bufb[t], sb[t]).wait()

    def wait_write(t):
        pltpu.make_async_copy(
            bufa[t], g_hbm.at[pl.ds(row0 * GC, GC)], sw[t]).wait()

    def step(j, t):
        ns = (t + 2) % 3

        @pl.when(jnp.logical_and(j >= 1, j + 2 < GPW))
        def _():
            wait_write(ns)

        @pl.when(j + 2 < GPW)
        def _():
            start(j + 2, ns)

        if with_d2:
            for v in range(4):
                sl = pl.ds(v * 16, 16)
                sv = sidx[j, sl]
                dv = didx[j, sl]
                si = [lax.shift_right_logical(sv, 7),
                      jnp.bitwise_and(sv, 127)]
                di = [lax.shift_right_logical(dv, 7),
                      jnp.bitwise_and(dv, 127)]
                dx = plsc.load_gather(posx, si) - plsc.load_gather(posx, di)
                dy = plsc.load_gather(posy, si) - plsc.load_gather(posy, di)
                dz = plsc.load_gather(posz, si) - plsc.load_gather(posz, di)
                d2buf[pl.ds(j * GC + v * 16, 16)] = dx * dx + dy * dy + dz * dz

        wait_gather(t)

        def addrow(r4, carry2):
            for u in range(4):
                for v in range(8):
                    sl = pl.ds(v * 16, 16)
                    r = r4 * 4 + u
                    bufa[t][r, sl] = bufa[t][r, sl] + bufb[t][r, sl]
            return carry2

        lax.fori_loop(0, GC // 4, addrow, 0)
        pltpu.async_copy(bufa[t],
                         g_hbm.at[pl.ds((row0 + j) * GC, GC)], sw[t])

    start(0, 0)
    start(1, 1)

    def body3(i, carry):
        for t in (0, 1, 2):
            step(i * 3 + t, t)
        return carry

    lax.fori_loop(0, GPW // 3, body3, 0)
    step(GPW - 1, (GPW - 1) % 3)
    if with_d2:
        pltpu.sync_copy(d2buf, d2_hbm.at[pl.ds(row0 * GC, GPW * GC)])
    wait_write(1)
    wait_write(2)
    wait_write(0)


def _make_gather(with_d2):
    outs = [jax.ShapeDtypeStruct((EPAD, D), jnp.float32)]
    scratch = [
        pltpu.VMEM((GPW, GC), jnp.int32),
        pltpu.VMEM((GPW, GC), jnp.int32),
        pltpu.VMEM((GC, D), jnp.float32),
        pltpu.VMEM((GC, D), jnp.float32),
        pltpu.VMEM((GC, D), jnp.float32),
        pltpu.VMEM((GC, D), jnp.float32),
        pltpu.VMEM((GC, D), jnp.float32),
        pltpu.VMEM((GC, D), jnp.float32),
    ]
    if with_d2:
        outs.append(jax.ShapeDtypeStruct((EPAD,), jnp.float32))
        scratch += [
            pltpu.VMEM((NPAD // CHUNK, CHUNK), jnp.float32),
            pltpu.VMEM((NPAD // CHUNK, CHUNK), jnp.float32),
            pltpu.VMEM((NPAD // CHUNK, CHUNK), jnp.float32),
            pltpu.VMEM((GPW * GC,), jnp.float32),
        ]
    scratch += [pltpu.SemaphoreType.DMA] * 9
    return pl.kernel(
        functools.partial(_gather_body, with_d2),
        out_type=tuple(outs),
        mesh=_sc_mesh,
        scratch_types=scratch,
        compiler_params=pltpu.CompilerParams(needs_layout_passes=False),
    )


_gather_d2 = _make_gather(True)
_gather = _make_gather(False)


def _scatter_body(m_hbm, dsts_hbm, out_hbm, didx, mb0, mb1, agg, sr0, sr1):
    c = lax.axis_index("c")
    s = lax.axis_index("s")
    mbuf = (mb0, mb1)
    sr = (sr0, sr1)

    def zrow(r, carry):
        for v in range(8):
            mb0[r, pl.ds(v * 16, 16)] = jnp.zeros((16,), jnp.float32)
        return carry

    lax.fori_loop(0, CHUNK, zrow, 0)
    for t in range(NPT // CHUNK):
        pltpu.sync_copy(mb0, agg.at[pl.ds(s * NPT + t * CHUNK, CHUNK)])
    plsc.subcore_barrier()

    row0 = c * RPC + s * RPW
    pltpu.sync_copy(dsts_hbm.at[pl.ds(row0, RPW)], didx)

    def start(j, t):
        pltpu.async_copy(m_hbm.at[pl.ds((row0 + j) * CHUNK, CHUNK)],
                         mbuf[t], sr[t])

    def wait_read(t):
        pltpu.make_async_copy(m_hbm.at[pl.ds(row0 * CHUNK, CHUNK)],
                              mbuf[t], sr[t]).wait()

    start(0, 0)

    def body2(i, carry):
        for t in (0, 1):
            j = i * 2 + t

            @pl.when(j + 1 < RPW)
            def _():
                start(j + 1, 1 - t)

            wait_read(t)
            pltpu.sync_copy(mbuf[t], agg.at[didx.at[j]], add=True)
        return carry

    lax.fori_loop(0, RPW // 2, body2, 0)
    plsc.subcore_barrier()
    pltpu.sync_copy(agg.at[pl.ds(s * NPT, NPT)],
                    out_hbm.at[c, pl.ds(s * NPT, NPT)])


_scatter = pl.kernel(
    _scatter_body,
    out_type=jax.ShapeDtypeStruct((NC, NPAD, D), jnp.float32),
    mesh=_sc_mesh,
    scratch_types=[
        pltpu.VMEM((RPW, CHUNK), jnp.int32),
        pltpu.VMEM((CHUNK, D), jnp.float32),
        pltpu.VMEM((CHUNK, D), jnp.float32),
        pltpu.VMEM_SHARED((NPAD, D), jnp.float32),
        pltpu.SemaphoreType.DMA,
        pltpu.SemaphoreType.DMA,
    ],
    compiler_params=pltpu.CompilerParams(needs_layout_passes=False),
)


def _silu(x):
    return x * jax.nn.sigmoid(x)


def _mm2_body(h_ref, wa_ref, wb_ref, a_ref, b_ref):
    h = h_ref[...]
    a_ref[...] = jnp.dot(h, wa_ref[...], preferred_element_type=jnp.float32)
    b_ref[...] = jnp.dot(h, wb_ref[...], preferred_element_type=jnp.float32)


def _mm2(h, wa, wb):
    bn = 512
    grid = NPAD // bn
    return pl.pallas_call(
        _mm2_body,
        grid=(grid,),
        in_specs=[
            pl.BlockSpec((bn, D), lambda i: (i, 0)),
            pl.BlockSpec((D, D), lambda i: (0, 0)),
            pl.BlockSpec((D, D), lambda i: (0, 0)),
        ],
        out_specs=[
            pl.BlockSpec((bn, D), lambda i: (i, 0)),
            pl.BlockSpec((bn, D), lambda i: (i, 0)),
        ],
        out_shape=[
            jax.ShapeDtypeStruct((NPAD, D), jnp.float32),
            jax.ShapeDtypeStruct((NPAD, D), jnp.float32),
        ],
    )(h, wa, wb)


def _msg_body(g_ref, ee_ref, wd_ref, we2_ref, be2_ref, out_ref):
    z = g_ref[...] + jnp.dot(ee_ref[...], wd_ref[...],
                             preferred_element_type=jnp.float32)
    m = _silu(z)
    y = jnp.dot(m, we2_ref[...], preferred_element_type=jnp.float32) + be2_ref[...]
    out_ref[...] = _silu(y)


def _msg(g, ee, wd, we2, be2):
    be = 1024
    grid = EPAD // be
    ec = ee.shape[1]
    return pl.pallas_call(
        _msg_body,
        grid=(grid,),
        in_specs=[
            pl.BlockSpec((be, D), lambda i: (i, 0)),
            pl.BlockSpec((be, ec), lambda i: (i, 0)),
            pl.BlockSpec((ec, D), lambda i: (0, 0)),
            pl.BlockSpec((D, D), lambda i: (0, 0)),
            pl.BlockSpec((1, D), lambda i: (0, 0)),
        ],
        out_specs=pl.BlockSpec((be, D), lambda i: (i, 0)),
        out_shape=jax.ShapeDtypeStruct((EPAD, D), jnp.float32),
    )(g, ee, wd, we2, be2)


def _upd_body(next_ab, *refs):
    if next_ab:
        (h_ref, p0_ref, p1_ref, w1h_ref, w1a_ref, b1_ref, w2_ref, b2_ref,
         wa_ref, wb_ref, hn_ref, a_ref, b_ref) = refs
    else:
        (h_ref, p0_ref, p1_ref, w1h_ref, w1a_ref, b1_ref, w2_ref, b2_ref,
         hn_ref) = refs
    h = h_ref[...]
    agg = p0_ref[...] + p1_ref[...]
    t = (jnp.dot(h, w1h_ref[...], preferred_element_type=jnp.float32)
         + jnp.dot(agg, w1a_ref[...], preferred_element_type=jnp.float32)
         + b1_ref[...])
    t = _silu(t)
    hn = jnp.dot(t, w2_ref[...], preferred_element_type=jnp.float32) + b2_ref[...]
    hn_ref[...] = hn
    if next_ab:
        a_ref[...] = jnp.dot(hn, wa_ref[...], preferred_element_type=jnp.float32)
        b_ref[...] = jnp.dot(hn, wb_ref[...], preferred_element_type=jnp.float32)


def _update(h, p0, p1, w1h, w1a, b1, w2, b2, wa=None, wb=None):
    bn = 512
    grid = NPAD // bn
    next_ab = wa is not None
    blk = pl.BlockSpec((bn, D), lambda i: (i, 0))
    wblk = pl.BlockSpec((D, D), lambda i: (0, 0))
    bblk = pl.BlockSpec((1, D), lambda i: (0, 0))
    in_specs = [blk, blk, blk, wblk, wblk, bblk, wblk, bblk]
    args = [h, p0, p1, w1h, w1a, b1, w2, b2]
    out_specs = [blk]
    out_shape = [jax.ShapeDtypeStruct((NPAD, D), jnp.float32)]
    if next_ab:
        in_specs += [wblk, wblk]
        args += [wa, wb]
        out_specs += [blk, blk]
        out_shape += [jax.ShapeDtypeStruct((NPAD, D), jnp.float32),
                      jax.ShapeDtypeStruct((NPAD, D), jnp.float32)]
    return pl.pallas_call(
        functools.partial(_upd_body, next_ab),
        grid=(grid,),
        in_specs=in_specs,
        out_specs=out_specs,
        out_shape=out_shape,
    )(*args)


def _pool_body(h_ref, bi_ref, wm1_ref, bm1_ref, wm2_ref, bm2_ref, wl_ref,
               bl_ref, out_ref, sum_ref, cnt_ref):
    i = pl.program_id(0)

    @pl.when(i == 0)
    def _():
        sum_ref[...] = jnp.zeros_like(sum_ref)
        cnt_ref[...] = jnp.zeros_like(cnt_ref)

    bi = bi_ref[...]
    oh = (bi == lax.broadcasted_iota(jnp.int32, bi.shape, 1)).astype(jnp.float32)
    dn = (((0,), (0,)), ((), ()))
    sum_ref[...] += lax.dot_general(oh, h_ref[...], dn,
                                    preferred_element_type=jnp.float32)
    cnt_ref[...] += lax.dot_general(oh, jnp.ones_like(h_ref[...]), dn,
                                    preferred_element_type=jnp.float32)

    @pl.when(i == pl.num_programs(0) - 1)
    def _():
        pooled = sum_ref[...] / jnp.maximum(cnt_ref[...], 1.0)
        r = jnp.maximum(
            jnp.dot(pooled, wm1_ref[...], preferred_element_type=jnp.float32)
            + bm1_ref[...], 0.0)
        r = jnp.maximum(
            jnp.dot(r, wm2_ref[...], preferred_element_type=jnp.float32)
            + bm2_ref[...], 0.0)
        out_ref[...] = (jnp.dot(r, wl_ref[...], preferred_element_type=jnp.float32)
                        + bl_ref[...])


def _pool_head(h, bi16, wm1, bm1, wm2, bm2, wlp, blp):
    bn = 1024
    grid = NPAD // bn
    wblk = pl.BlockSpec((D, D), lambda i: (0, 0))
    bblk = pl.BlockSpec((1, D), lambda i: (0, 0))
    return pl.pallas_call(
        _pool_body,
        grid=(grid,),
        in_specs=[
            pl.BlockSpec((bn, D), lambda i: (i, 0)),
            pl.BlockSpec((bn, NB), lambda i: (i, 0)),
            wblk, bblk, wblk, bblk, wblk, bblk,
        ],
        out_specs=pl.BlockSpec((NB, D), lambda i: (0, 0)),
        out_shape=jax.ShapeDtypeStruct((NB, D), jnp.float32),
        scratch_shapes=[
            pltpu.VMEM((NB, D), jnp.float32),
            pltpu.VMEM((NB, D), jnp.float32),
        ],
    )(h, bi16, wm1, bm1, wm2, bm2, wlp, blp)


def kernel(x, pos, edge_index, edge_attr, batch_indices,
           We1_0, be1_0, We2_0, be2_0, Wh1_0, bh1_0, Wh2_0, bh2_0,
           We1_1, be1_1, We2_1, be2_1, Wh1_1, bh1_1, Wh2_1, bh2_1,
           Wm1, bm1, Wm2, bm2, Wl, bl):
    f32 = jnp.float32
    xp = jnp.pad(x, ((0, NPAD - N), (0, 0)))
    px = jnp.pad(pos[:, 0], (0, NPAD - N)).reshape(NPAD // CHUNK, CHUNK)
    py = jnp.pad(pos[:, 1], (0, NPAD - N)).reshape(NPAD // CHUNK, CHUNK)
    pz = jnp.pad(pos[:, 2], (0, NPAD - N)).reshape(NPAD // CHUNK, CHUNK)
    src = jnp.pad(edge_index[0], (0, EPAD - E)).reshape(GROWS, GC)
    dst = jnp.pad(edge_index[1], (0, EPAD - E)).reshape(GROWS, GC)
    dsts = jnp.pad(edge_index[1], (0, EPAD - E),
                   constant_values=N).reshape(ROWS, CHUNK)
    eap = jnp.pad(edge_attr, ((0, EPAD - E), (0, 0)))
    bi16 = jnp.broadcast_to(
        jnp.pad(batch_indices, (0, NPAD - N), constant_values=NB)[:, None],
        (NPAD, NB))

    wd0 = jnp.concatenate([We1_0[2 * D + 1:], We1_0[2 * D:2 * D + 1],
                           be1_0[None]], axis=0)
    wd1 = jnp.concatenate([We1_1[2 * D + 1:], We1_1[2 * D:2 * D + 1],
                           be1_1[None]], axis=0)
    wlp = jnp.pad(Wl, ((0, 0), (0, D - 1)))
    blp = jnp.pad(bl, (0, D - 1))[None]

    a0, b0 = _mm2(xp, We1_0[:D], We1_0[D:2 * D])
    g0, d2 = _gather_d2(a0, b0, px, py, pz, src, dst)
    ee = jnp.concatenate([eap, d2[:, None], jnp.ones((EPAD, 1), f32)], axis=1)
    m0 = _msg(g0, ee, wd0, We2_0, be2_0[None])
    p = _scatter(m0, dsts)
    h1, a1, b1 = _update(xp, p[0], p[1], Wh1_0[:D], Wh1_0[D:], bh1_0[None],
                         Wh2_0, bh2_0[None], We1_1[:D], We1_1[D:2 * D])
    g1, = _gather(a1, b1, src, dst)
    m1 = _msg(g1, ee, wd1, We2_1, be2_1[None])
    p = _scatter(m1, dsts)
    h2 = _update(h1, p[0], p[1], Wh1_1[:D], Wh1_1[D:], bh1_1[None],
                 Wh2_1, bh2_1[None])[0]
    out = _pool_head(h2, bi16, Wm1, bm1[None], Wm2, bm2[None], wlp, blp)
    return out[:, :1]
